# initial kernel scaffold (unmeasured)
import jax
import jax.numpy as jnp
from jax import lax
from jax.experimental import pallas as pl
from jax.experimental.pallas import tpu as pltpu


def kernel(
    x,
):
    def body(*refs):
        pass

    out_shape = jax.ShapeDtypeStruct(..., jnp.float32)
    return pl.pallas_call(body, out_shape=out_shape)(...)


# baseline (device time: 83260 ns/iter reference)
import jax
import jax.numpy as jnp
from jax import lax
from jax.experimental import pallas as pl
from jax.experimental.pallas import tpu as pltpu

N_DEV = 4
M_PER = 1024
M_GLOBAL = N_DEV * M_PER


def _stage(val, k, j, flip):
    m = val.shape[0]
    i = lax.broadcasted_iota(jnp.int32, (m, 1), 0)
    b = (i & j) == 0
    a = (i & k) == 0
    up = jnp.concatenate([val[j:], val[:j]], axis=0)
    down = jnp.concatenate([val[m - j :], val[: m - j]], axis=0)
    partner = jnp.where(b, up, down)
    keep_min = a == b
    if flip is not None:
        keep_min = keep_min != flip
    return jnp.where(keep_min, jnp.minimum(val, partner), jnp.maximum(val, partner))


def kernel(x):
    m_per, n = x.shape
    assert (m_per, n) == (M_PER, 256), (m_per, n)

    def body(x_ref, out_ref, gath_ref, comm_ref, send_sems, recv_sems):
        my_pos = lax.axis_index("i")
        left = (my_pos - 1) % N_DEV
        right = (my_pos + 1) % N_DEV

        barrier_sem = pltpu.get_barrier_semaphore()
        for nbr in (left, right):
            pl.semaphore_signal(
                barrier_sem, inc=1,
                device_id=(nbr,), device_id_type=pl.DeviceIdType.MESH,
            )
        pl.semaphore_wait(barrier_sem, 2)

        desc = (my_pos % 2) == 1
        val = x_ref[:, :]
        for k_log in range(1, 11):
            k = 1 << k_log
            for j_log in range(k_log - 1, -1, -1):
                val = _stage(val, k, 1 << j_log, desc)

        gath_ref[pl.ds(my_pos * M_PER, M_PER), :] = val
        comm_ref[0, :, :] = val

        for h in range(N_DEV - 1):
            send_slot = h % 2
            recv_slot = (h + 1) % 2
            rdma = pltpu.make_async_remote_copy(
                src_ref=comm_ref.at[send_slot],
                dst_ref=comm_ref.at[recv_slot],
                send_sem=send_sems.at[send_slot],
                recv_sem=recv_sems.at[recv_slot],
                device_id=(right,),
                device_id_type=pl.DeviceIdType.MESH,
            )
            rdma.start()
            rdma.wait()
            origin = (my_pos - h - 1) % N_DEV
            gath_ref[pl.ds(origin * M_PER, M_PER), :] = comm_ref[recv_slot, :, :]

        gv = gath_ref[:, :]
        for k_log in (11, 12):
            k = 1 << k_log
            for j_log in range(k_log - 1, -1, -1):
                gv = _stage(gv, k, 1 << j_log, None)
        gath_ref[:, :] = gv
        out_ref[:, :] = gath_ref[pl.ds(my_pos * M_PER, M_PER), :]

    return pl.pallas_call(
        body,
        out_shape=jax.ShapeDtypeStruct((M_PER, n), x.dtype),
        in_specs=[pl.BlockSpec(memory_space=pltpu.VMEM)],
        out_specs=pl.BlockSpec(memory_space=pltpu.VMEM),
        scratch_shapes=[
            pltpu.VMEM((M_GLOBAL, n), x.dtype),
            pltpu.VMEM((2, M_PER, n), x.dtype),
            pltpu.SemaphoreType.DMA((2,)),
            pltpu.SemaphoreType.DMA((2,)),
        ],
        compiler_params=pltpu.CompilerParams(collective_id=0),
    )(x)


# device time: 66426 ns/iter; 1.2534x vs baseline; 1.2534x over previous
import jax
import jax.numpy as jnp
from jax import lax
from jax.experimental import pallas as pl
from jax.experimental.pallas import tpu as pltpu

N_DEV = 4
M_PER = 1024


def _stage(val, k, j, flip):
    m = val.shape[0]
    i = lax.broadcasted_iota(jnp.int32, (m, 1), 0)
    b = (i & j) == 0
    a = (i & k) == 0
    up = jnp.concatenate([val[j:], val[:j]], axis=0)
    down = jnp.concatenate([val[m - j :], val[: m - j]], axis=0)
    partner = jnp.where(b, up, down)
    keep_min = a == b
    if flip is not None:
        keep_min = keep_min != flip
    return jnp.where(keep_min, jnp.minimum(val, partner), jnp.maximum(val, partner))


def _combine(val, theirs, keep_min):
    return jnp.where(keep_min, jnp.minimum(val, theirs), jnp.maximum(val, theirs))


def kernel(x):
    m_per, n = x.shape
    assert (m_per, n) == (M_PER, 256), (m_per, n)

    def body(x_ref, out_ref, cur_ref, ex_ref, send_sems, recv_sems):
        m = lax.axis_index("i")
        b = m ^ (m >> 1)

        barrier_sem = pltpu.get_barrier_semaphore()
        for nbr in (m ^ 1, 3 - m):
            pl.semaphore_signal(
                barrier_sem, inc=1,
                device_id=(nbr,), device_id_type=pl.DeviceIdType.MESH,
            )
        pl.semaphore_wait(barrier_sem, 2)

        def exchange(phase, partner, val):
            cur_ref[:, :] = val
            rdma = pltpu.make_async_remote_copy(
                src_ref=cur_ref,
                dst_ref=ex_ref.at[phase],
                send_sem=send_sems.at[phase],
                recv_sem=recv_sems.at[phase],
                device_id=(partner,),
                device_id_type=pl.DeviceIdType.MESH,
            )
            rdma.start()
            rdma.wait()
            return ex_ref[phase, :, :]

        val = x_ref[:, :]
        flip_local = (b & 1) == 1
        for k_log in range(1, 11):
            for j_log in range(k_log - 1, -1, -1):
                val = _stage(val, 1 << k_log, 1 << j_log, flip_local)

        theirs = exchange(0, m ^ 1, val)
        val = _combine(val, theirs, ((b & 1) == 0) == ((b & 2) == 0))

        flip1 = (b & 2) != 0
        for j_log in range(9, -1, -1):
            val = _stage(val, 2048, 1 << j_log, flip1)

        theirs = exchange(1, 3 - m, val)
        val = _combine(val, theirs, (b & 2) == 0)

        theirs = exchange(2, m ^ 1, val)
        val = _combine(val, theirs, (m & 1) == 0)

        for j_log in range(9, -1, -1):
            val = _stage(val, 4096, 1 << j_log, None)

        out_ref[:, :] = val

    return pl.pallas_call(
        body,
        out_shape=jax.ShapeDtypeStruct((M_PER, n), x.dtype),
        in_specs=[pl.BlockSpec(memory_space=pltpu.VMEM)],
        out_specs=pl.BlockSpec(memory_space=pltpu.VMEM),
        scratch_shapes=[
            pltpu.VMEM((M_PER, n), x.dtype),
            pltpu.VMEM((3, M_PER, n), x.dtype),
            pltpu.SemaphoreType.DMA((3,)),
            pltpu.SemaphoreType.DMA((3,)),
        ],
        compiler_params=pltpu.CompilerParams(collective_id=0),
    )(x)


# device time: 48147 ns/iter; 1.7293x vs baseline; 1.3796x over previous
import jax
import jax.numpy as jnp
from jax import lax
from jax.experimental import pallas as pl
from jax.experimental.pallas import tpu as pltpu

N_DEV = 4
M_PER = 1024
N_COLS = 256
HALF = N_COLS // 2


def _stage(val, k, j, flip):
    m = val.shape[0]
    i = lax.broadcasted_iota(jnp.int32, (m, 1), 0)
    b = (i & j) == 0
    a = (i & k) == 0
    up = jnp.concatenate([val[j:], val[:j]], axis=0)
    down = jnp.concatenate([val[m - j :], val[: m - j]], axis=0)
    partner = jnp.where(b, up, down)
    keep_min = a == b
    if flip is not None:
        keep_min = keep_min != flip
    return jnp.where(keep_min, jnp.minimum(val, partner), jnp.maximum(val, partner))


def _combine(val, theirs, keep_min):
    return jnp.where(keep_min, jnp.minimum(val, theirs), jnp.maximum(val, theirs))


def _sort_local(val, flip):
    for k_log in range(1, 11):
        for j_log in range(k_log - 1, -1, -1):
            val = _stage(val, 1 << k_log, 1 << j_log, flip)
    return val


def _local_merge(val, k, flip):
    for j_log in range(9, -1, -1):
        val = _stage(val, k, 1 << j_log, flip)
    return val


def kernel(x):
    m_per, n = x.shape
    assert (m_per, n) == (M_PER, N_COLS), (m_per, n)

    def body(x_ref, out_ref, stage_ref, ex_ref, send_sems, recv_sems):
        m = lax.axis_index("i")
        b = m ^ (m >> 1)

        barrier_sem = pltpu.get_barrier_semaphore()
        for nbr in (m ^ 1, 3 - m):
            pl.semaphore_signal(
                barrier_sem, inc=1,
                device_id=(nbr,), device_id_type=pl.DeviceIdType.MESH,
            )
        pl.semaphore_wait(barrier_sem, 2)

        def start_exchange(phase, h, partner, val):
            stage_ref[phase, h, :, :] = val
            rdma = pltpu.make_async_remote_copy(
                src_ref=stage_ref.at[phase, h],
                dst_ref=ex_ref.at[phase, h],
                send_sem=send_sems.at[phase, h],
                recv_sem=recv_sems.at[phase, h],
                device_id=(partner,),
                device_id_type=pl.DeviceIdType.MESH,
            )
            rdma.start()
            return rdma

        def finish_exchange(rdma, phase, h):
            rdma.wait()
            return ex_ref[phase, h, :, :]

        flip_local = (b & 1) == 1
        km1 = ((b & 1) == 0) == ((b & 2) == 0)
        km2 = (b & 2) == 0
        km3 = (m & 1) == 0
        flip1 = (b & 2) != 0

        va = _sort_local(x_ref[:, :HALF], flip_local)
        e1a = start_exchange(0, 0, m ^ 1, va)
        vb = _sort_local(x_ref[:, HALF:], flip_local)
        e1b = start_exchange(0, 1, m ^ 1, vb)

        va = _combine(va, finish_exchange(e1a, 0, 0), km1)
        va = _local_merge(va, 2048, flip1)
        e2a = start_exchange(1, 0, 3 - m, va)

        vb = _combine(vb, finish_exchange(e1b, 0, 1), km1)
        vb = _local_merge(vb, 2048, flip1)
        e2b = start_exchange(1, 1, 3 - m, vb)

        va = _combine(va, finish_exchange(e2a, 1, 0), km2)
        e3a = start_exchange(2, 0, m ^ 1, va)
        vb = _combine(vb, finish_exchange(e2b, 1, 1), km2)
        e3b = start_exchange(2, 1, m ^ 1, vb)

        va = _combine(va, finish_exchange(e3a, 2, 0), km3)
        out_ref[:, :HALF] = _local_merge(va, 4096, None)
        vb = _combine(vb, finish_exchange(e3b, 2, 1), km3)
        out_ref[:, HALF:] = _local_merge(vb, 4096, None)

    return pl.pallas_call(
        body,
        out_shape=jax.ShapeDtypeStruct((M_PER, n), x.dtype),
        in_specs=[pl.BlockSpec(memory_space=pltpu.VMEM)],
        out_specs=pl.BlockSpec(memory_space=pltpu.VMEM),
        scratch_shapes=[
            pltpu.VMEM((3, 2, M_PER, HALF), x.dtype),
            pltpu.VMEM((3, 2, M_PER, HALF), x.dtype),
            pltpu.SemaphoreType.DMA((3, 2)),
            pltpu.SemaphoreType.DMA((3, 2)),
        ],
        compiler_params=pltpu.CompilerParams(collective_id=0),
    )(x)


# device time: 41955 ns/iter; 1.9845x vs baseline; 1.1476x over previous
import jax
import jax.numpy as jnp
from jax import lax
from jax.experimental import pallas as pl
from jax.experimental.pallas import tpu as pltpu

N_DEV = 4
M_PER = 1024
N_COLS = 256
HALF = N_COLS // 2
CH = 256
NCH = 4


def _stage_c(val, c, k, j, flip):
    il = lax.broadcasted_iota(jnp.int32, (CH, 1), 0)
    b = (il & j) == 0
    if k <= 128:
        keep_min = ((il & k) == 0) == b
    elif k == 256:
        keep_min = b if (c & 1) == 0 else jnp.logical_not(b)
    elif k == 512:
        keep_min = b if (c & 2) == 0 else jnp.logical_not(b)
    else:
        keep_min = b
    if flip is not None:
        keep_min = keep_min != flip
    up = jnp.concatenate([val[j:], val[:j]], axis=0)
    down = jnp.concatenate([val[CH - j :], val[: CH - j]], axis=0)
    partner = jnp.where(b, up, down)
    return jnp.where(keep_min, jnp.minimum(val, partner), jnp.maximum(val, partner))


def _cross(v_lo, v_hi, flip):
    lo = jnp.minimum(v_lo, v_hi)
    hi = jnp.maximum(v_lo, v_hi)
    if flip is None:
        return lo, hi
    return jnp.where(flip, hi, lo), jnp.where(flip, lo, hi)


def _tail8(val, c, k, flip):
    for j_log in range(7, -1, -1):
        val = _stage_c(val, c, k, 1 << j_log, flip)
    return val


def _sort_pre(v, flip):
    notflip = jnp.logical_not(flip)
    for k_log in range(1, 8):
        for j_log in range(k_log - 1, -1, -1):
            v = [_stage_c(v[c], c, 1 << k_log, 1 << j_log, flip) for c in range(NCH)]
    for j_log in range(7, -1, -1):
        v = [_stage_c(v[c], c, 256, 1 << j_log, flip) for c in range(NCH)]
    v[0], v[1] = _cross(v[0], v[1], flip)
    v[2], v[3] = _cross(v[2], v[3], notflip)
    for j_log in range(7, -1, -1):
        v = [_stage_c(v[c], c, 512, 1 << j_log, flip) for c in range(NCH)]
    v[0], v[2] = _cross(v[0], v[2], flip)
    v[1], v[3] = _cross(v[1], v[3], flip)
    v[0], v[1] = _cross(v[0], v[1], flip)
    v[2], v[3] = _cross(v[2], v[3], flip)
    return v


def _combine(val, theirs, keep_min):
    return jnp.where(keep_min, jnp.minimum(val, theirs), jnp.maximum(val, theirs))


def kernel(x):
    m_per, n = x.shape
    assert (m_per, n) == (M_PER, N_COLS), (m_per, n)

    def body(x_ref, out_ref, stage_ref, ex_ref, send_sems, recv_sems):
        m = lax.axis_index("i")
        b = m ^ (m >> 1)

        barrier_sem = pltpu.get_barrier_semaphore()
        for nbr in (m ^ 1, 3 - m):
            pl.semaphore_signal(
                barrier_sem, inc=1,
                device_id=(nbr,), device_id_type=pl.DeviceIdType.MESH,
            )
        pl.semaphore_wait(barrier_sem, 2)

        def rdma(phase, h, c, partner):
            return pltpu.make_async_remote_copy(
                src_ref=stage_ref.at[phase, h, c],
                dst_ref=ex_ref.at[phase, h, c],
                send_sem=send_sems.at[phase, h, c],
                recv_sem=recv_sems.at[phase, h, c],
                device_id=(partner,),
                device_id_type=pl.DeviceIdType.MESH,
            )

        def send(phase, h, c, partner, val):
            stage_ref[phase, h, c, :, :] = val
            r = rdma(phase, h, c, partner)
            r.start()
            return r

        def recv(r, phase, h, c):
            r.wait()
            return ex_ref[phase, h, c, :, :]

        flip_local = (b & 1) == 1
        km1 = ((b & 1) == 0) == ((b & 2) == 0)
        km2 = (b & 2) == 0
        km3 = (m & 1) == 0
        flip1 = (b & 2) != 0

        def chunks_of(col_slice):
            return [x_ref[c * CH : (c + 1) * CH, col_slice] for c in range(NCH)]

        va = _sort_pre(chunks_of(slice(0, HALF)), flip_local)
        e1a = []
        for c in range(NCH):
            va[c] = _tail8(va[c], c, 1024, flip_local)
            e1a.append(send(0, 0, c, m ^ 1, va[c]))
        vb = _sort_pre(chunks_of(slice(HALF, N_COLS)), flip_local)
        e1b = []
        for c in range(NCH):
            vb[c] = _tail8(vb[c], c, 1024, flip_local)
            e1b.append(send(0, 1, c, m ^ 1, vb[c]))

        def c_phase(v, e, phase_in, h, k, flip, phase_out, partner, km):
            for c in range(NCH):
                v[c] = _combine(v[c], recv(e[c], phase_in, h, c), km)
            v[0], v[2] = _cross(v[0], v[2], flip)
            v[1], v[3] = _cross(v[1], v[3], flip)
            v[0], v[1] = _cross(v[0], v[1], flip)
            v[2], v[3] = _cross(v[2], v[3], flip)
            out = []
            for c in range(NCH):
                v[c] = _tail8(v[c], c, k, flip)
                out.append(send(phase_out, h, c, partner, v[c]))
            return out

        e2a = c_phase(va, e1a, 0, 0, 2048, flip1, 1, 3 - m, km1)
        e2b = c_phase(vb, e1b, 0, 1, 2048, flip1, 1, 3 - m, km1)

        e3a, e3b = [], []
        for c in range(NCH):
            va[c] = _combine(va[c], recv(e2a[c], 1, 0, c), km2)
            e3a.append(send(2, 0, c, m ^ 1, va[c]))
        for c in range(NCH):
            vb[c] = _combine(vb[c], recv(e2b[c], 1, 1, c), km2)
            e3b.append(send(2, 1, c, m ^ 1, vb[c]))

        def finish(v, e, h, col0):
            for c in range(NCH):
                v[c] = _combine(v[c], recv(e[c], 2, h, c), km3)
            v[0], v[2] = _cross(v[0], v[2], None)
            v[1], v[3] = _cross(v[1], v[3], None)
            v[0], v[1] = _cross(v[0], v[1], None)
            v[2], v[3] = _cross(v[2], v[3], None)
            for c in range(NCH):
                out_ref[c * CH : (c + 1) * CH, col0 : col0 + HALF] = _tail8(
                    v[c], c, 4096, None
                )

        finish(va, e3a, 0, 0)
        finish(vb, e3b, 1, HALF)

    return pl.pallas_call(
        body,
        out_shape=jax.ShapeDtypeStruct((M_PER, n), x.dtype),
        in_specs=[pl.BlockSpec(memory_space=pltpu.VMEM)],
        out_specs=pl.BlockSpec(memory_space=pltpu.VMEM),
        scratch_shapes=[
            pltpu.VMEM((3, 2, NCH, CH, HALF), x.dtype),
            pltpu.VMEM((3, 2, NCH, CH, HALF), x.dtype),
            pltpu.SemaphoreType.DMA((3, 2, NCH)),
            pltpu.SemaphoreType.DMA((3, 2, NCH)),
        ],
        compiler_params=pltpu.CompilerParams(collective_id=0),
    )(x)
